# baseline (speedup 1.0000x reference)
import functools

import jax
import jax.numpy as jnp
from jax import lax
from jax.experimental import pallas as pl
from jax.experimental.pallas import tpu as pltpu
from jax.experimental.pallas import tpu_sc as plsc

_NC = 2
_NS = 16
_NW = _NC * _NS
_LANES = 16


def _make_sc_add(B, L, D, dtype):
    lpw = L // _NW
    ch = 16
    n_chunks = lpw // ch
    nvec = D // _LANES
    assert L % _NW == 0 and lpw % ch == 0 and D % _LANES == 0

    mesh = plsc.VectorSubcoreMesh(core_axis_name="c", subcore_axis_name="s")

    @functools.partial(
        pl.kernel,
        out_type=jax.ShapeDtypeStruct((B, L, D), dtype),
        mesh=mesh,
        scratch_types=[
            pltpu.VMEM((ch, D), dtype),
            pltpu.VMEM((B * ch, D), dtype),
        ],
    )
    def sc_add(x_hbm, pos_hbm, out_hbm, pbuf, xbuf):
        wid = lax.axis_index("s") * _NC + lax.axis_index("c")
        for c in range(n_chunks):
            l0 = wid * lpw + c * ch
            pltpu.sync_copy(pos_hbm.at[pl.ds(l0, ch)], pbuf)
            for b in range(B):
                pltpu.sync_copy(x_hbm.at[b, pl.ds(l0, ch)],
                                xbuf.at[pl.ds(b * ch, ch)])

            def row_body(r, _):
                pr = lax.rem(r, ch)
                for k in range(nvec):
                    v = pbuf[pr, pl.ds(k * _LANES, _LANES)]
                    plsc.addupdate(xbuf.at[r, pl.ds(k * _LANES, _LANES)], v)
                return ()

            lax.fori_loop(0, B * ch, row_body, ())
            for b in range(B):
                pltpu.sync_copy(xbuf.at[pl.ds(b * ch, ch)],
                                out_hbm.at[b, pl.ds(l0, ch)])

    return sc_add


def kernel(inputs, pos_table):
    B, L, D = inputs.shape
    return _make_sc_add(B, L, D, inputs.dtype)(inputs, pos_table)


# speedup vs baseline: 1.3162x; 1.3162x over previous
import functools

import jax
import jax.numpy as jnp
from jax import lax
from jax.experimental import pallas as pl
from jax.experimental.pallas import tpu as pltpu
from jax.experimental.pallas import tpu_sc as plsc

_NC = 2
_NS = 16
_NW = _NC * _NS
_LANES = 16

_L_SC = 1536


def _tc_add_block(x_ref, p_ref, o_ref):
    o_ref[...] = x_ref[...] + p_ref[...]


def _tc_add(inputs, pos_table):
    B, L, D = inputs.shape
    bl = 512
    return pl.pallas_call(
        _tc_add_block,
        grid=(L // bl, B),
        in_specs=[
            pl.BlockSpec((1, bl, D), lambda l, b: (b, l, 0)),
            pl.BlockSpec((bl, D), lambda l, b: (l, 0)),
        ],
        out_specs=pl.BlockSpec((1, bl, D), lambda l, b: (b, l, 0)),
        out_shape=jax.ShapeDtypeStruct((B, L, D), inputs.dtype),
    )(inputs, pos_table)


def _make_sc_add(B, L, D, dtype):
    lpw = L // _NW
    ch = 16
    n_chunks = lpw // ch
    nvec = D // _LANES
    assert L % _NW == 0 and lpw % ch == 0 and D % _LANES == 0

    mesh = plsc.VectorSubcoreMesh(core_axis_name="c", subcore_axis_name="s")

    @functools.partial(
        pl.kernel,
        out_type=jax.ShapeDtypeStruct((B, L, D), dtype),
        mesh=mesh,
        scratch_types=[
            pltpu.VMEM((ch, D), dtype),
            pltpu.VMEM((B * ch, D), dtype),
        ],
    )
    def sc_add(x_hbm, pos_hbm, out_hbm, pbuf, xbuf):
        wid = lax.axis_index("s") * _NC + lax.axis_index("c")
        for c in range(n_chunks):
            l0 = wid * lpw + c * ch
            pltpu.sync_copy(pos_hbm.at[pl.ds(l0, ch)], pbuf)
            for b in range(B):
                pltpu.sync_copy(x_hbm.at[b, pl.ds(l0, ch)],
                                xbuf.at[pl.ds(b * ch, ch)])

            def row_body(r, _):
                pr = lax.rem(r, ch)
                for k in range(nvec):
                    v = pbuf[pr, pl.ds(k * _LANES, _LANES)]
                    plsc.addupdate(xbuf.at[r, pl.ds(k * _LANES, _LANES)], v)
                return ()

            lax.fori_loop(0, B * ch, row_body, ())
            for b in range(B):
                pltpu.sync_copy(xbuf.at[pl.ds(b * ch, ch)],
                                out_hbm.at[b, pl.ds(l0, ch)])

    return sc_add


def kernel(inputs, pos_table):
    B, L, D = inputs.shape
    l_tc = L - _L_SC
    out_tc = _tc_add(inputs[:, :l_tc], pos_table[:l_tc])
    out_sc = _make_sc_add(B, _L_SC, D, inputs.dtype)(
        inputs[:, l_tc:], pos_table[l_tc:])
    return jnp.concatenate([out_tc, out_sc], axis=1)


# speedup vs baseline: 2.0221x; 1.5363x over previous
import functools

import jax
import jax.numpy as jnp
from jax import lax
from jax.experimental import pallas as pl
from jax.experimental.pallas import tpu as pltpu
from jax.experimental.pallas import tpu_sc as plsc

_NC = 2
_NS = 16
_NW = _NC * _NS
_LANES = 16


def _make_sc_add(B, L, D, dtype):
    lpw = L // _NW
    ch = 16
    n_chunks = lpw // ch
    assert L % _NW == 0 and lpw % ch == 0 and D % _LANES == 0

    mesh = plsc.VectorSubcoreMesh(core_axis_name="c", subcore_axis_name="s")

    @functools.partial(
        pl.kernel,
        out_type=jax.ShapeDtypeStruct((B, L, D), dtype),
        mesh=mesh,
        scratch_types=[
            pltpu.VMEM((ch, D), dtype),
            pltpu.VMEM((B * ch, D), dtype),
        ],
    )
    def sc_add(x_hbm, pos_hbm, out_hbm, pbuf, xbuf):
        wid = lax.axis_index("s") * _NC + lax.axis_index("c")
        for c in range(n_chunks):
            l0 = wid * lpw + c * ch
            pltpu.sync_copy(pos_hbm.at[pl.ds(l0, ch)], pbuf)
            for b in range(B):
                pltpu.sync_copy(x_hbm.at[b, pl.ds(l0, ch)],
                                xbuf.at[pl.ds(b * ch, ch)])
            for b in range(B):
                pltpu.sync_copy(xbuf.at[pl.ds(b * ch, ch)],
                                out_hbm.at[b, pl.ds(l0, ch)])

    return sc_add


def kernel(inputs, pos_table):
    B, L, D = inputs.shape
    return _make_sc_add(B, L, D, inputs.dtype)(inputs, pos_table)


# speedup vs baseline: 4.5701x; 2.2601x over previous
import jax
import jax.numpy as jnp
from jax.experimental import pallas as pl


def _add_block(x_ref, p_ref, o_ref):
    o_ref[...] = x_ref[...] + p_ref[...]


def kernel(inputs, pos_table):
    B, L, D = inputs.shape
    BL = 2048

    return pl.pallas_call(
        _add_block,
        grid=(L // BL, B),
        in_specs=[
            pl.BlockSpec((1, BL, D), lambda l, b: (b, l, 0)),
            pl.BlockSpec((BL, D), lambda l, b: (l, 0)),
        ],
        out_specs=pl.BlockSpec((1, BL, D), lambda l, b: (b, l, 0)),
        out_shape=jax.ShapeDtypeStruct((B, L, D), inputs.dtype),
    )(inputs, pos_table)
